# Initial kernel scaffold; baseline (speedup 1.0000x reference)
import jax
import jax.numpy as jnp
from jax.experimental import pallas as pl


def kernel(pred, target):
    raise NotImplementedError("write your pallas kernel here")


# speedup vs baseline: 1.3026x; 1.3026x over previous
import jax
import jax.numpy as jnp
from jax.experimental import pallas as pl
from jax.experimental.pallas import tpu as pltpu

_W = 0.1
_NBINS = 64
_EPS = 1e-08
_LOG2E = 1.4426950408889634


def _kl_body(pred_ref, targ_ref, out_ref, ys_p, ys_t, hist_p, hist_t):
    t = targ_ref[0]
    p = pred_ref[0]

    vmin = jnp.min(t)
    vmax = jnp.max(t)
    denom = vmax - vmin + _EPS
    w = 1.0 / _NBINS
    s = jnp.sqrt(jnp.float32(_LOG2E)) / (jnp.sqrt(jnp.float32(2.0)) * w * denom)

    ys_t[...] = t * s
    ys_p[...] = p * s

    def bin_body(b, _):
        c = (b.astype(jnp.float32) + 0.5) * w
        mbs = (vmin + c * denom) * s
        yt = ys_t[...]
        yp = ys_p[...]
        et = jnp.exp2((yt - mbs) * (mbs - yt))
        ep = jnp.exp2((yp - mbs) * (mbs - yp))
        hist_t[pl.ds(b, 1), :] = jnp.sum(et, axis=0, keepdims=True)
        hist_p[pl.ds(b, 1), :] = jnp.sum(ep, axis=0, keepdims=True)
        return 0

    jax.lax.fori_loop(0, _NBINS, bin_body, 0)

    ht = jnp.sum(hist_t[...], axis=1, keepdims=True)
    hp = jnp.sum(hist_p[...], axis=1, keepdims=True)
    tp = ht / (jnp.sum(ht) + _EPS)
    pp = hp / (jnp.sum(hp) + _EPS)
    kl = jnp.sum(tp * (jnp.log(tp + _EPS) - jnp.log(pp + _EPS)))
    out_ref[0] = jnp.full((8, 128), kl, dtype=jnp.float32)


def kernel(pred, target):
    B = pred.shape[0]
    n = pred.size // B
    lanes = 128
    rows = n // lanes
    p3 = pred.reshape(B, rows, lanes)
    t3 = target.reshape(B, rows, lanes)

    out = pl.pallas_call(
        _kl_body,
        out_shape=jax.ShapeDtypeStruct((B, 8, 128), jnp.float32),
        grid=(B,),
        in_specs=[
            pl.BlockSpec((1, rows, lanes), lambda i: (i, 0, 0)),
            pl.BlockSpec((1, rows, lanes), lambda i: (i, 0, 0)),
        ],
        out_specs=pl.BlockSpec((1, 8, 128), lambda i: (i, 0, 0)),
        scratch_shapes=[
            pltpu.VMEM((rows, lanes), jnp.float32),
            pltpu.VMEM((rows, lanes), jnp.float32),
            pltpu.VMEM((_NBINS, 128), jnp.float32),
            pltpu.VMEM((_NBINS, 128), jnp.float32),
        ],
        compiler_params=pltpu.CompilerParams(
            dimension_semantics=("parallel",),
        ),
        name="kl_soft_hist",
    )(p3, t3)

    return _W * jnp.mean(out[:, 0, 0])
